# baseline (speedup 1.0000x reference)
import functools

import jax
import jax.numpy as jnp
from jax import lax
from jax.experimental import pallas as pl
from jax.experimental.pallas import tpu as pltpu
from jax.experimental.pallas import tpu_sc as plsc

V = 1000000
D = 32
B = 16384
GAMMA = 0.5
NW = 32
BPW = B // NW
NCHUNK = BPW // 128
NBLK = BPW // 16


def _vrsqrt(x):
    i = plsc.bitcast(x, jnp.int32)
    i = jnp.int32(0x5F3759DF) - (i >> 1)
    y = plsc.bitcast(i, jnp.float32)
    half_x = 0.5 * x
    for _ in range(3):
        y = y * (1.5 - half_x * y * y)
    return y


def _sc_kernel(idx_hbm, p_hbm, c_hbm, wb_hbm,
               rating_hbm, scale_hbm, dir_hbm,
               idx_v, rows_p, rows_c, vbuf, wb_v,
               scale_v, rating_v, dir_v, sem):
    wid = lax.axis_index("s") * 2 + lax.axis_index("c")
    base = wid * BPW

    pltpu.sync_copy(idx_hbm.at[pl.ds(wid * NCHUNK, NCHUNK)], idx_v)
    pltpu.sync_copy(wb_hbm, wb_v)

    copies = []
    for j in range(NCHUNK):
        copies.append(pltpu.async_copy(
            p_hbm.at[idx_v.at[j]], rows_p.at[pl.ds(j * 128, 128)], sem))
        copies.append(pltpu.async_copy(
            c_hbm.at[idx_v.at[j]], rows_c.at[pl.ds(j * 128, 128)], sem))
    for cp in copies:
        cp.wait()

    def blk_body(blk, carry):
        base16 = blk * 16
        row_ids = base16 + lax.iota(jnp.int32, 16)
        acc_c2 = jnp.full((16,), 0.0, jnp.float32)
        acc_v2 = jnp.full((16,), 0.0, jnp.float32)
        for d in range(D):
            cd = jnp.full((16,), d, jnp.int32)
            pc = plsc.load_gather(rows_p, [row_ids, cd])
            cc = plsc.load_gather(rows_c, [row_ids, cd])
            vc = GAMMA * pc + cc
            acc_c2 = acc_c2 + cc * cc
            acc_v2 = acc_v2 + vc * vc
            vbuf[d] = vc
        scale = acc_c2 * _vrsqrt(jnp.maximum(acc_c2, jnp.float32(1e-30)))
        rinv = _vrsqrt(jnp.maximum(acc_v2, jnp.float32(1e-24)))
        logit = jnp.full((16,), 0.0, jnp.float32)
        for d in range(D):
            dirc = vbuf[d] * rinv
            plsc.store_scatter(dir_v, [row_ids, jnp.full((16,), d, jnp.int32)], dirc)
            logit = logit + dirc * wb_v[d]
        logit = scale * logit + wb_v[D]
        rating = 1.0 / (1.0 + jnp.exp(-logit))
        scale_v[pl.ds(base16, 16)] = scale
        rating_v[pl.ds(base16, 16)] = rating
        return carry

    lax.fori_loop(0, NBLK, blk_body, 0)

    pltpu.sync_copy(rating_v, rating_hbm.at[pl.ds(base, BPW)])
    pltpu.sync_copy(scale_v, scale_hbm.at[pl.ds(base, BPW)])
    pltpu.sync_copy(dir_v, dir_hbm.at[pl.ds(base, BPW)])


@jax.jit
def _run(idx2, P, C, wb):
    mesh = plsc.VectorSubcoreMesh(core_axis_name="c", subcore_axis_name="s")
    k = functools.partial(
        pl.kernel, mesh=mesh,
        compiler_params=pltpu.CompilerParams(
            needs_layout_passes=False, use_tc_tiling_on_sc=False),
        out_type=(
            jax.ShapeDtypeStruct((B,), jnp.float32),
            jax.ShapeDtypeStruct((B,), jnp.float32),
            jax.ShapeDtypeStruct((B, D), jnp.float32),
        ),
        scratch_types=[
            pltpu.VMEM((NCHUNK, 128), jnp.int32),
            pltpu.VMEM((BPW, D), jnp.float32),
            pltpu.VMEM((BPW, D), jnp.float32),
            pltpu.VMEM((D, 16), jnp.float32),
            pltpu.VMEM((D + 1, 16), jnp.float32),
            pltpu.VMEM((BPW,), jnp.float32),
            pltpu.VMEM((BPW,), jnp.float32),
            pltpu.VMEM((BPW, D), jnp.float32),
            pltpu.SemaphoreType.DMA,
        ],
    )(_sc_kernel)
    return k(idx2, P, C, wb)


def kernel(item_indices, P, C, W, b):
    idx2 = item_indices.reshape(B // 128, 128).astype(jnp.int32)
    wb = jnp.concatenate(
        [jnp.broadcast_to(W.reshape(D, 1), (D, 16)),
         jnp.broadcast_to(b.reshape(1, 1), (1, 16))], axis=0)
    rating, scale, direction = _run(idx2, P, C, wb)
    return (rating.reshape(B, 1), scale.reshape(B, 1), direction)


# speedup vs baseline: 3.3720x; 3.3720x over previous
import functools

import jax
import jax.numpy as jnp
from jax import lax
from jax.experimental import pallas as pl
from jax.experimental.pallas import tpu as pltpu
from jax.experimental.pallas import tpu_sc as plsc

V = 1000000
D = 32
B = 16384
GAMMA = 0.5
NW = 32
BPW = B // NW
NBLK = BPW // 16
HK = 8


def _vrsqrt(x):
    i = plsc.bitcast(x, jnp.int32)
    i = jnp.int32(0x5F3759DF) - (i >> 1)
    y = plsc.bitcast(i, jnp.float32)
    half_x = 0.5 * x
    for _ in range(3):
        y = y * (1.5 - half_x * y * y)
    return y


def _sc_kernel(idx_hbm, pt_hbm, ct_hbm, wb_hbm,
               rating_hbm, scale_hbm, dirt_hbm,
               idx_v, tiles_p, tiles_c, blk_p, blk_c, wb_v,
               scale_v, rating_v, dirt_v, sem):
    wid = lax.axis_index("s") * 2 + lax.axis_index("c")
    base = wid * BPW

    pltpu.sync_copy(idx_hbm.at[wid], idx_v)
    pltpu.sync_copy(wb_hbm, wb_v)

    d_ids = lax.iota(jnp.int32, 16)
    wlow = wb_v[0, pl.ds(0, 16)]
    whigh = wb_v[0, pl.ds(16, 16)]
    bias = jnp.full((16,), 1.0, jnp.float32) * wb_v[1, pl.ds(0, 16)][0]

    def blk_body(blk, carry):
        b16 = blk * 16
        vec = idx_v[pl.ds(b16, 16)]
        jvec = (vec >> 7) * 128
        cvec = vec & 127

        for h in range(2):
            cps = []
            for l8 in range(HK):
                l = h * HK + l8
                j = pl.multiple_of(jvec[l], 128)
                for dblk in range(4):
                    cps.append(pltpu.async_copy(
                        pt_hbm.at[pl.ds(dblk * 8, 8), pl.ds(j, 128)],
                        tiles_p.at[l8, pl.ds(dblk * 8, 8), :], sem))
                    cps.append(pltpu.async_copy(
                        ct_hbm.at[pl.ds(dblk * 8, 8), pl.ds(j, 128)],
                        tiles_c.at[l8, pl.ds(dblk * 8, 8), :], sem))
            for cp in cps:
                cp.wait()
            for l8 in range(HK):
                l = h * HK + l8
                cl = jnp.full((16,), 0, jnp.int32) + cvec[l]
                lsplat = jnp.full((16,), l, jnp.int32)
                for half in range(2):
                    rows = d_ids + (half * 16)
                    pcol = plsc.load_gather(tiles_p, [jnp.full((16,), l8, jnp.int32), rows, cl])
                    ccol = plsc.load_gather(tiles_c, [jnp.full((16,), l8, jnp.int32), rows, cl])
                    plsc.store_scatter(blk_p, [rows, lsplat], pcol)
                    plsc.store_scatter(blk_c, [rows, lsplat], ccol)

        acc_c2 = jnp.full((16,), 0.0, jnp.float32)
        acc_v2 = jnp.full((16,), 0.0, jnp.float32)
        for d in range(D):
            pc = blk_p[d, pl.ds(0, 16)]
            cc = blk_c[d, pl.ds(0, 16)]
            vc = GAMMA * pc + cc
            acc_c2 = acc_c2 + cc * cc
            acc_v2 = acc_v2 + vc * vc
            dirt_v[d, pl.ds(b16, 16)] = vc
        scale = acc_c2 * _vrsqrt(jnp.maximum(acc_c2, jnp.float32(1e-30)))
        rinv = _vrsqrt(jnp.maximum(acc_v2, jnp.float32(1e-24)))
        logit = jnp.full((16,), 0.0, jnp.float32)
        for d in range(D):
            dirc = dirt_v[d, pl.ds(b16, 16)] * rinv
            dirt_v[d, pl.ds(b16, 16)] = dirc
            wsrc = wlow if d < 16 else whigh
            wd = jnp.full((16,), 1.0, jnp.float32) * wsrc[d % 16]
            logit = logit + dirc * wd
        logit = scale * logit + bias
        rating = 1.0 / (1.0 + jnp.exp(-logit))
        scale_v[pl.ds(b16, 16)] = scale
        rating_v[pl.ds(b16, 16)] = rating
        return carry

    lax.fori_loop(0, NBLK, blk_body, 0)

    pltpu.sync_copy(rating_v, rating_hbm.at[pl.ds(base, BPW)])
    pltpu.sync_copy(scale_v, scale_hbm.at[pl.ds(base, BPW)])
    pltpu.sync_copy(dirt_v, dirt_hbm.at[:, pl.ds(base, BPW)])


@jax.jit
def _run(idx2, Pt, Ct, wb):
    mesh = plsc.VectorSubcoreMesh(core_axis_name="c", subcore_axis_name="s")
    k = functools.partial(
        pl.kernel, mesh=mesh,
        compiler_params=pltpu.CompilerParams(
            needs_layout_passes=False, use_tc_tiling_on_sc=True),
        out_type=(
            jax.ShapeDtypeStruct((B,), jnp.float32),
            jax.ShapeDtypeStruct((B,), jnp.float32),
            jax.ShapeDtypeStruct((D, B), jnp.float32),
        ),
        scratch_types=[
            pltpu.VMEM((BPW,), jnp.int32),
            pltpu.VMEM((HK, D, 128), jnp.float32),
            pltpu.VMEM((HK, D, 128), jnp.float32),
            pltpu.VMEM((D, 128), jnp.float32),
            pltpu.VMEM((D, 128), jnp.float32),
            pltpu.VMEM((8, 128), jnp.float32),
            pltpu.VMEM((BPW,), jnp.float32),
            pltpu.VMEM((BPW,), jnp.float32),
            pltpu.VMEM((D, BPW), jnp.float32),
            pltpu.SemaphoreType.DMA,
        ],
    )(_sc_kernel)
    return k(idx2, Pt, Ct, wb)


def kernel(item_indices, P, C, W, b):
    idx2 = item_indices.reshape(NW, BPW).astype(jnp.int32)
    wb = jnp.zeros((8, 128), jnp.float32)
    wb = wb.at[0, :D].set(W.reshape(D))
    wb = wb.at[1, 0].set(b[0])
    rating, scale, dirt = _run(idx2, P.T, C.T, wb)
    return (rating.reshape(B, 1), scale.reshape(B, 1), dirt.T)


# speedup vs baseline: 3.3987x; 1.0079x over previous
import functools

import jax
import jax.numpy as jnp
from jax import lax
from jax.experimental import pallas as pl
from jax.experimental.pallas import tpu as pltpu
from jax.experimental.pallas import tpu_sc as plsc

V = 1000000
D = 32
B = 16384
GAMMA = 0.5
NW = 32
BPW = B // NW
NBLK = BPW // 16
HK = 8


def _vrsqrt(x):
    i = plsc.bitcast(x, jnp.int32)
    i = jnp.int32(0x5F3759DF) - (i >> 1)
    y = plsc.bitcast(i, jnp.float32)
    half_x = 0.5 * x
    for _ in range(3):
        y = y * (1.5 - half_x * y * y)
    return y


def _sc_kernel(idx_hbm, pt_hbm, ct_hbm, wb_hbm,
               rating_hbm, scale_hbm, dirt_hbm,
               idx_v, tiles_p, tiles_c, tiles_p2, tiles_c2, blk_p, blk_c,
               wb_v, scale_v, rating_v, dirt_v, sem, sem2):
    wid = lax.axis_index("s") * 2 + lax.axis_index("c")
    base = wid * BPW

    pltpu.sync_copy(idx_hbm.at[wid], idx_v)
    pltpu.sync_copy(wb_hbm, wb_v)

    d_ids = lax.iota(jnp.int32, 16)
    wlow = wb_v[0, pl.ds(0, 16)]
    whigh = wb_v[0, pl.ds(16, 16)]
    bias = jnp.full((16,), 1.0, jnp.float32) * wb_v[1, pl.ds(0, 16)][0]

    def blk_body(blk, carry):
        b16 = blk * 16
        vec = idx_v[pl.ds(b16, 16)]
        jvec = (vec >> 7) * 128
        cvec = vec & 127

        def fire(q, t_p, t_c, s):
            cps = []
            for l4 in range(4):
                j = pl.multiple_of(jvec[q * 4 + l4], 128)
                cps.append(pltpu.async_copy(
                    pt_hbm.at[:, pl.ds(j, 128)], t_p.at[l4], s))
                cps.append(pltpu.async_copy(
                    ct_hbm.at[:, pl.ds(j, 128)], t_c.at[l4], s))
            return cps

        def extract(q, t_p, t_c):
            for l4 in range(4):
                l = q * 4 + l4
                cl = jnp.full((16,), 0, jnp.int32) + cvec[l]
                lsplat = jnp.full((16,), l, jnp.int32)
                l4s = jnp.full((16,), l4, jnp.int32)
                for half in range(2):
                    rows = d_ids + (half * 16)
                    pcol = plsc.load_gather(t_p, [l4s, rows, cl])
                    ccol = plsc.load_gather(t_c, [l4s, rows, cl])
                    plsc.store_scatter(blk_p, [rows, lsplat], pcol)
                    plsc.store_scatter(blk_c, [rows, lsplat], ccol)

        sets = ((tiles_p, tiles_c, sem), (tiles_p2, tiles_c2, sem2))
        pend = fire(0, *sets[0])
        for q in range(4):
            nxt = fire(q + 1, *sets[(q + 1) % 2]) if q < 3 else []
            for cp in pend:
                cp.wait()
            extract(q, sets[q % 2][0], sets[q % 2][1])
            pend = nxt

        acc_c2 = jnp.full((16,), 0.0, jnp.float32)
        acc_v2 = jnp.full((16,), 0.0, jnp.float32)
        for d in range(D):
            pc = blk_p[d, pl.ds(0, 16)]
            cc = blk_c[d, pl.ds(0, 16)]
            vc = GAMMA * pc + cc
            acc_c2 = acc_c2 + cc * cc
            acc_v2 = acc_v2 + vc * vc
            dirt_v[d, pl.ds(b16, 16)] = vc
        scale = acc_c2 * _vrsqrt(jnp.maximum(acc_c2, jnp.float32(1e-30)))
        rinv = _vrsqrt(jnp.maximum(acc_v2, jnp.float32(1e-24)))
        logit = jnp.full((16,), 0.0, jnp.float32)
        for d in range(D):
            dirc = dirt_v[d, pl.ds(b16, 16)] * rinv
            dirt_v[d, pl.ds(b16, 16)] = dirc
            wsrc = wlow if d < 16 else whigh
            wd = jnp.full((16,), 1.0, jnp.float32) * wsrc[d % 16]
            logit = logit + dirc * wd
        logit = scale * logit + bias
        rating = 1.0 / (1.0 + jnp.exp(-logit))
        scale_v[pl.ds(b16, 16)] = scale
        rating_v[pl.ds(b16, 16)] = rating
        return carry

    lax.fori_loop(0, NBLK, blk_body, 0)

    pltpu.sync_copy(rating_v, rating_hbm.at[pl.ds(base, BPW)])
    pltpu.sync_copy(scale_v, scale_hbm.at[pl.ds(base, BPW)])
    pltpu.sync_copy(dirt_v, dirt_hbm.at[:, pl.ds(base, BPW)])


@jax.jit
def _run(idx2, Pt, Ct, wb):
    mesh = plsc.VectorSubcoreMesh(core_axis_name="c", subcore_axis_name="s")
    k = functools.partial(
        pl.kernel, mesh=mesh,
        compiler_params=pltpu.CompilerParams(
            needs_layout_passes=False, use_tc_tiling_on_sc=True),
        out_type=(
            jax.ShapeDtypeStruct((B,), jnp.float32),
            jax.ShapeDtypeStruct((B,), jnp.float32),
            jax.ShapeDtypeStruct((D, B), jnp.float32),
        ),
        scratch_types=[
            pltpu.VMEM((BPW,), jnp.int32),
            pltpu.VMEM((4, D, 128), jnp.float32),
            pltpu.VMEM((4, D, 128), jnp.float32),
            pltpu.VMEM((4, D, 128), jnp.float32),
            pltpu.VMEM((4, D, 128), jnp.float32),
            pltpu.VMEM((D, 128), jnp.float32),
            pltpu.VMEM((D, 128), jnp.float32),
            pltpu.VMEM((8, 128), jnp.float32),
            pltpu.VMEM((BPW,), jnp.float32),
            pltpu.VMEM((BPW,), jnp.float32),
            pltpu.VMEM((D, BPW), jnp.float32),
            pltpu.SemaphoreType.DMA,
            pltpu.SemaphoreType.DMA,
        ],
    )(_sc_kernel)
    return k(idx2, Pt, Ct, wb)


def kernel(item_indices, P, C, W, b):
    idx2 = item_indices.reshape(NW, BPW).astype(jnp.int32)
    wb = jnp.zeros((8, 128), jnp.float32)
    wb = wb.at[0, :D].set(W.reshape(D))
    wb = wb.at[1, 0].set(b[0])
    rating, scale, dirt = _run(idx2, P.T, C.T, wb)
    return (rating.reshape(B, 1), scale.reshape(B, 1), dirt.T)
